# baseline (speedup 1.0000x reference)
import functools

import jax
import jax.numpy as jnp
import numpy as np
from jax import lax
from jax.experimental import pallas as pl
from jax.experimental.pallas import tpu as pltpu
from jax.experimental.pallas import tpu_sc as plsc

VOCAB = 1000000
D = 64
BATCH = 4096
SEQ = 200
TOKENS = BATCH * SEQ

NC = 2
NS = 16
NW = NC * NS
LANES = 16

TOK_PER_W = TOKENS // NW
SEQ_PER_CHUNK = 2
CHUNK = SEQ_PER_CHUNK * SEQ
N_CHUNKS = TOK_PER_W // CHUNK
IDX_MINOR = 100
IDX_ROWS = CHUNK // IDX_MINOR
X_ROWS = TOKENS // IDX_MINOR


def _compute_encoding(max_len, d):
    enc = np.zeros((max_len, d), dtype=np.float32)
    pos = np.arange(0, max_len, dtype=np.float32)
    for i in range(d // 2):
        enc[:, 2 * i] = np.sin(pos / 10000 ** (2 * i / d))
        enc[:, 2 * i + 1] = np.cos(pos / 10000 ** (2 * i / d))
    return enc


_ENC = _compute_encoding(SEQ, D)


def _body(x_hbm, table_hbm, enc_hbm, out_hbm, idx_v, rows_v, enc_v, sem):
    wid = lax.axis_index("s") * NC + lax.axis_index("c")

    pltpu.sync_copy(enc_hbm, enc_v)

    def chunk_body(c, _):
        xrow = wid * (TOK_PER_W // IDX_MINOR) + c * IDX_ROWS
        tok = wid * TOK_PER_W + c * CHUNK

        pltpu.sync_copy(x_hbm.at[pl.ds(xrow, IDX_ROWS)], idx_v)
        for g in range(IDX_ROWS):
            pltpu.async_copy(
                table_hbm.at[idx_v.at[g]],
                rows_v.at[pl.ds(g * IDX_MINOR, IDX_MINOR)],
                sem,
            ).wait()

        def add_body(s, _):
            for j in range(D // LANES):
                sl = pl.ds(j * LANES, LANES)
                e = enc_v[s, sl]
                for q in range(SEQ_PER_CHUNK):
                    rows_v[q * SEQ + s, sl] += e
            return ()

        lax.fori_loop(0, SEQ, add_body, (), unroll=False)

        pltpu.sync_copy(rows_v, out_hbm.at[pl.ds(tok, CHUNK)])
        return ()

    lax.fori_loop(0, N_CHUNKS, chunk_body, (), unroll=False)


@jax.jit
def kernel(x, table):
    x2d = x.reshape(X_ROWS, IDX_MINOR).astype(jnp.int32)
    enc = jnp.asarray(_ENC)
    mesh = plsc.VectorSubcoreMesh(core_axis_name="c", subcore_axis_name="s")
    out = pl.kernel(
        _body,
        out_type=jax.ShapeDtypeStruct((TOKENS, D), jnp.float32),
        mesh=mesh,
        compiler_params=pltpu.CompilerParams(use_tc_tiling_on_sc=False),
        scratch_types=[
            pltpu.VMEM((IDX_ROWS, IDX_MINOR), jnp.int32),
            pltpu.VMEM((CHUNK, D), jnp.float32),
            pltpu.VMEM((SEQ, D), jnp.float32),
            pltpu.SemaphoreType.DMA,
        ],
    )(x2d, table, enc)
    return out.reshape(BATCH, SEQ, D)


# speedup vs baseline: 1.2024x; 1.2024x over previous
import functools

import jax
import jax.numpy as jnp
import numpy as np
from jax import lax
from jax.experimental import pallas as pl
from jax.experimental.pallas import tpu as pltpu
from jax.experimental.pallas import tpu_sc as plsc

VOCAB = 1000000
D = 64
BATCH = 4096
SEQ = 200
TOKENS = BATCH * SEQ

NC = 2
NS = 16
NW = NC * NS
LANES = 16

TOK_PER_W = TOKENS // NW
SEQ_PER_CHUNK = 2
CHUNK = SEQ_PER_CHUNK * SEQ
N_CHUNKS = TOK_PER_W // CHUNK
IDX_MINOR = 100
IDX_ROWS = CHUNK // IDX_MINOR
X_ROWS = TOKENS // IDX_MINOR


def _compute_encoding(max_len, d):
    enc = np.zeros((max_len, d), dtype=np.float32)
    pos = np.arange(0, max_len, dtype=np.float32)
    for i in range(d // 2):
        enc[:, 2 * i] = np.sin(pos / 10000 ** (2 * i / d))
        enc[:, 2 * i + 1] = np.cos(pos / 10000 ** (2 * i / d))
    return enc


_ENC = _compute_encoding(SEQ, D)


def _body(x_hbm, table_hbm, enc_hbm, out_hbm,
          idx_v, rows_v, enc_v, idx_sem, gat_sem, out_sem):
    wid = lax.axis_index("s") * NC + lax.axis_index("c")
    xrow0 = wid * (TOK_PER_W // IDX_MINOR)
    tok0 = wid * TOK_PER_W

    pltpu.sync_copy(enc_hbm, enc_v)

    def idx_start(c, b):
        pltpu.make_async_copy(
            x_hbm.at[pl.ds(xrow0 + c * IDX_ROWS, IDX_ROWS)],
            idx_v.at[b], idx_sem.at[b]).start()

    def idx_wait(b):
        pltpu.make_async_copy(
            x_hbm.at[pl.ds(0, IDX_ROWS)], idx_v.at[b], idx_sem.at[b]).wait()

    def gather_start(b):
        for g in range(IDX_ROWS):
            pltpu.make_async_copy(
                table_hbm.at[idx_v.at[b].at[g]],
                rows_v.at[b].at[pl.ds(g * IDX_MINOR, IDX_MINOR)],
                gat_sem.at[b]).start()

    def gather_wait(b):
        pltpu.make_async_copy(
            out_hbm.at[pl.ds(0, CHUNK)], rows_v.at[b], gat_sem.at[b]).wait()

    def out_start(c, b):
        pltpu.make_async_copy(
            rows_v.at[b], out_hbm.at[pl.ds(tok0 + c * CHUNK, CHUNK)],
            out_sem.at[b]).start()

    def out_wait(b):
        pltpu.make_async_copy(
            rows_v.at[b], out_hbm.at[pl.ds(0, CHUNK)], out_sem.at[b]).wait()

    idx_start(0, 0)
    idx_wait(0)
    gather_start(0)
    idx_start(1, 1)

    def pair_body(c2, _):
        for b in (0, 1):
            c = 2 * c2 + b
            nb = 1 - b

            @pl.when(c >= 1)
            def _():
                out_wait(nb)

            @pl.when(c + 1 < N_CHUNKS)
            def _():
                idx_wait(nb)
                gather_start(nb)

            gather_wait(b)

            @pl.when(c + 2 < N_CHUNKS)
            def _():
                idx_start(c + 2, b)

            def add_body(s, _):
                for j in range(D // LANES):
                    sl = pl.ds(j * LANES, LANES)
                    e = enc_v[s, sl]
                    for q in range(SEQ_PER_CHUNK):
                        rows_v[b, q * SEQ + s, sl] += e
                return ()

            lax.fori_loop(0, SEQ, add_body, (), unroll=False)

            out_start(c, b)
        return ()

    lax.fori_loop(0, N_CHUNKS // 2, pair_body, (), unroll=False)

    out_wait((N_CHUNKS - 1) % 2)


@jax.jit
def kernel(x, table):
    x2d = x.reshape(X_ROWS, IDX_MINOR).astype(jnp.int32)
    enc = jnp.asarray(_ENC)
    mesh = plsc.VectorSubcoreMesh(core_axis_name="c", subcore_axis_name="s")
    out = pl.kernel(
        _body,
        out_type=jax.ShapeDtypeStruct((TOKENS, D), jnp.float32),
        mesh=mesh,
        compiler_params=pltpu.CompilerParams(use_tc_tiling_on_sc=False),
        scratch_types=[
            pltpu.VMEM((2, IDX_ROWS, IDX_MINOR), jnp.int32),
            pltpu.VMEM((2, CHUNK, D), jnp.float32),
            pltpu.VMEM((SEQ, D), jnp.float32),
            pltpu.SemaphoreType.DMA((2,)),
            pltpu.SemaphoreType.DMA((2,)),
            pltpu.SemaphoreType.DMA((2,)),
        ],
    )(x2d, table, enc)
    return out.reshape(BATCH, SEQ, D)
